# baseline (speedup 1.0000x reference)
import jax
import jax.numpy as jnp
from jax import lax
from jax.experimental import pallas as pl
from jax.experimental.pallas import tpu as pltpu

_N_VARS = 512
_N_CLUSTER = 64
_SEQ_LEN = 4096
_HIDDEN = 1024
_BATCH = 32
_SEQ_BLK = 2048


def _bf16_dot(wb, xm):
    hi = xm.astype(jnp.bfloat16)
    r1 = xm - hi.astype(jnp.float32)
    lo = r1.astype(jnp.bfloat16)
    r2 = r1 - lo.astype(jnp.float32)
    lo2 = r2.astype(jnp.bfloat16)
    dims = (((1,), (0,)), ((), ()))
    acc = lax.dot_general(wb, hi, dims, preferred_element_type=jnp.float32)
    acc += lax.dot_general(wb, lo, dims, preferred_element_type=jnp.float32)
    acc += lax.dot_general(wb, lo2, dims, preferred_element_type=jnp.float32)
    return acc


def _fused_kernel(x0_ref, x1_ref, x2_ref, x3_ref, w_ref, b_ref, c_ref,
                  out_ref, acc_ref, et_ref):
    bidx = pl.program_id(0)
    quarter = _SEQ_LEN // 4

    for j, xr in enumerate((x0_ref, x1_ref, x2_ref, x3_ref)):
        xb = xr[0].astype(jnp.bfloat16).astype(jnp.float32)
        sl = slice(j * quarter, (j + 1) * quarter)

        @pl.when(bidx == 0)
        def _init(sl=sl, xb=xb):
            acc_ref[sl, :] = xb

        @pl.when(bidx > 0)
        def _accum(sl=sl, xb=xb):
            acc_ref[sl, :] += xb

    @pl.when(bidx == _BATCH - 1)
    def _head():
        for i in range(_SEQ_LEN // _SEQ_BLK):
            sl = slice(i * _SEQ_BLK, (i + 1) * _SEQ_BLK)
            part = _bf16_dot(w_ref[:, sl], acc_ref[sl, :])
            if i == 0:
                et_ref[...] = part
            else:
                et_ref[...] += part
        et = et_ref[...] * (1.0 / _BATCH) + b_ref[...]
        norm = jnp.sqrt(jnp.sum(et * et, axis=0, keepdims=True))
        en = (et / jnp.maximum(norm, 1e-12)).astype(jnp.bfloat16)
        c = c_ref[...]
        cnorm = jnp.sqrt(jnp.sum(c * c, axis=1, keepdims=True))
        cn = (c / jnp.maximum(cnorm, 1e-12)).astype(jnp.bfloat16)
        st = lax.dot_general(
            cn, en,
            dimension_numbers=(((1,), (0,)), ((), ())),
            preferred_element_type=jnp.float32,
        )
        m = jnp.max(st, axis=0, keepdims=True)
        iota_k = lax.broadcasted_iota(jnp.int32, (_N_CLUSTER, _N_VARS), 0)
        masked = jnp.where(st >= m, iota_k, _N_CLUSTER)
        idx = jnp.min(masked, axis=0, keepdims=True)
        pt = (iota_k == idx).astype(jnp.float32)
        r = lax.broadcasted_iota(jnp.int32, (_N_VARS, _N_VARS), 0)
        q = lax.broadcasted_iota(jnp.int32, (_N_VARS, _N_VARS), 1)
        eye = (r == q).astype(jnp.float32)
        out_ref[...] = lax.dot_general(
            eye, pt,
            dimension_numbers=(((1,), (1,)), ((), ())),
            preferred_element_type=jnp.float32,
        )


def kernel(x, W, b, centroids):
    wb16 = W.astype(jnp.bfloat16)
    b2 = b.reshape(_HIDDEN, 1)
    return pl.pallas_call(
        _fused_kernel,
        grid=(_BATCH,),
        in_specs=[
            pl.BlockSpec((1, _SEQ_LEN // 4, _N_VARS), lambda i: (i, 0, 0)),
            pl.BlockSpec((1, _SEQ_LEN // 4, _N_VARS), lambda i: (i, 1, 0)),
            pl.BlockSpec((1, _SEQ_LEN // 4, _N_VARS), lambda i: (i, 2, 0)),
            pl.BlockSpec((1, _SEQ_LEN // 4, _N_VARS), lambda i: (i, 3, 0)),
            pl.BlockSpec((_HIDDEN, _SEQ_LEN), lambda i: (0, 0)),
            pl.BlockSpec((_HIDDEN, 1), lambda i: (0, 0)),
            pl.BlockSpec((_N_CLUSTER, _HIDDEN), lambda i: (0, 0)),
        ],
        out_specs=pl.BlockSpec((_N_VARS, _N_CLUSTER), lambda i: (0, 0)),
        out_shape=jax.ShapeDtypeStruct((_N_VARS, _N_CLUSTER), jnp.float32),
        scratch_shapes=[
            pltpu.VMEM((_SEQ_LEN, _N_VARS), jnp.float32),
            pltpu.VMEM((_HIDDEN, _N_VARS), jnp.float32),
        ],
    )(x, x, x, x, wb16, b2, centroids)
